# Initial kernel scaffold; baseline (speedup 1.0000x reference)
import jax
import jax.numpy as jnp
from jax.experimental import pallas as pl


def kernel(encoder_output, mask, W_ih, W_hh, b_ih, b_hh, W_fc, b_fc):
    raise NotImplementedError("write your pallas kernel here")


# speedup vs baseline: 39.2674x; 39.2674x over previous
import jax
import jax.numpy as jnp
from jax.experimental import pallas as pl

B, T, ENC, HID = 16, 64, 1024, 1024


def _loss_kernel(x_ref, mask_ref, wx_ref, out_ref):
    x = x_ref[...]
    wx = wx_ref[...]
    bk = jnp.sum(x * wx[None, :, :], axis=-1)
    mask = mask_ref[...]
    lengths = jnp.sum(mask, axis=1, keepdims=True)
    kpos = jax.lax.broadcasted_iota(jnp.int32, (B, T), 1)
    m = jnp.max(bk, axis=1, keepdims=True)
    e = jnp.where(kpos < lengths, jnp.exp(bk - m), 0.0)
    kk = jax.lax.broadcasted_iota(jnp.int32, (T, T), 0)
    ss = jax.lax.broadcasted_iota(jnp.int32, (T, T), 1)
    tri = (kk >= ss).astype(jnp.float32)
    suf = jax.lax.dot(e, tri, precision=jax.lax.Precision.HIGHEST)
    valid = (kpos >= 3) & (kpos < lengths)
    lse = m + jnp.log(suf)
    term = jnp.where(valid, lse - bk, 0.0)
    total = jnp.sum(term)
    count = jnp.sum(valid.astype(jnp.float32))
    out_ref[...] = jnp.broadcast_to(total / count, (1, 1))


def kernel(encoder_output, mask, W_ih, W_hh, b_ih, b_hh, W_fc, b_fc):
    del W_ih, W_hh, b_ih, b_hh, b_fc
    wx = W_fc[:, HID:]
    out = pl.pallas_call(
        _loss_kernel,
        out_shape=jax.ShapeDtypeStruct((1, 1), jnp.float32),
    )(encoder_output, mask, wx)
    return out[0, 0]
